# Initial kernel scaffold; baseline (speedup 1.0000x reference)
import jax
import jax.numpy as jnp
from jax.experimental import pallas as pl


def kernel(x_user, x_item, edge_index_user_rates_item, edge_index_item_rated_by_user, edge_index_user_follows_user, W_neigh_ui, W_root_ui, b_ui, W_neigh_iu, W_root_iu, b_iu, W_neigh_uu, W_root_uu, b_uu):
    raise NotImplementedError("write your pallas kernel here")


# speedup vs baseline: 1.7544x; 1.7544x over previous
import functools

import jax
import jax.numpy as jnp
from jax import lax
from jax.experimental import pallas as pl
from jax.experimental.pallas import tpu as pltpu
from jax.experimental.pallas import tpu_sc as plsc

N_USER = 50000
N_ITEM = 50000
D = 128
NC = 2
NS = 16
NW = NC * NS
B = 128
CH = 32
NCHUNK = D // CH
NPAD = 50176
ROWS_PER_TILE = NPAD // NS
ZROWS = 112
NZCOPY = ROWS_PER_TILE // ZROWS
DRAIN = ROWS_PER_TILE // 2
PAD_DST = 50000

E_UI = 320000
E_UU = 160000
NB_UI = 80
NB_UU = 40
NBMAX = NB_UI


def _sc_body(xu0, xu1, xu2, xu3, xi0, xi1, xi2, xi3,
             s_ui, d_ui, s_iu, d_iu, s_uu, d_uu,
             agg_ui, deg_ui, agg_iu, deg_iu, agg_uu, deg_uu,
             acc, zbuf, gbuf, sidx, didx):
  c = lax.axis_index("c")
  s = lax.axis_index("s")
  w = c * NS + s

  zeros16 = jnp.zeros((16,), jnp.float32)
  onehot16 = jnp.where(lax.iota(jnp.int32, 16) == 0, 1.0, 0.0)

  @pl.loop(0, ZROWS)
  def _(i):
    zbuf[i, pl.ds(0, 16)] = zeros16
    zbuf[i, pl.ds(16, 16)] = zeros16

  rels = [
      ((xu0, xu1, xu2, xu3), s_ui, d_ui, agg_ui, deg_ui, NB_UI),
      ((xi0, xi1, xi2, xi3), s_iu, d_iu, agg_iu, deg_iu, NB_UI),
      ((xu0, xu1, xu2, xu3), s_uu, d_uu, agg_uu, deg_uu, NB_UU),
  ]

  for tabs, s_hbm, d_hbm, agg_out, deg_out, nb in rels:
    pltpu.sync_copy(s_hbm.at[w], sidx.at[pl.ds(0, nb)])
    pltpu.sync_copy(d_hbm.at[w], didx.at[pl.ds(0, nb)])

    for chunk in range(NCHUNK + 1):
      is_deg = chunk == NCHUNK
      @pl.loop(0, NZCOPY)
      def _(k):
        pltpu.sync_copy(zbuf, acc.at[pl.ds(s * ROWS_PER_TILE + k * ZROWS, ZROWS)])
      plsc.subcore_barrier()

      if is_deg:
        @pl.loop(0, B)
        def _(i):
          gbuf[i, pl.ds(0, 16)] = onehot16
          gbuf[i, pl.ds(16, 16)] = zeros16

        @pl.loop(0, nb)
        def _(j):
          pltpu.sync_copy(gbuf, acc.at[didx.at[j]], add=True)
      else:
        table = tabs[chunk]

        @pl.loop(0, nb)
        def _(j):
          pltpu.sync_copy(table.at[sidx.at[j]], gbuf)
          pltpu.sync_copy(gbuf, acc.at[didx.at[j]], add=True)

      plsc.subcore_barrier()
      for k in range(2):
        r0 = s * ROWS_PER_TILE + k * DRAIN
        src = acc.at[pl.ds(r0, DRAIN)]
        if is_deg:
          pltpu.sync_copy(src, deg_out.at[c, pl.ds(r0, DRAIN)])
        else:
          pltpu.sync_copy(src, agg_out.at[c, pl.ds(r0, DRAIN),
                                          pl.ds(chunk * CH, CH)])
      plsc.subcore_barrier()


def _sc_aggregate(xu_tabs, xi_tabs, idx_ui, idx_iu, idx_uu):
  mesh = plsc.VectorSubcoreMesh(core_axis_name="c", subcore_axis_name="s")
  f32 = jnp.float32
  out_type = [
      jax.ShapeDtypeStruct((NC, NPAD, D), f32),
      jax.ShapeDtypeStruct((NC, NPAD, CH), f32),
      jax.ShapeDtypeStruct((NC, NPAD, D), f32),
      jax.ShapeDtypeStruct((NC, NPAD, CH), f32),
      jax.ShapeDtypeStruct((NC, NPAD, D), f32),
      jax.ShapeDtypeStruct((NC, NPAD, CH), f32),
  ]
  scratch_types = [
      pltpu.VMEM_SHARED((NPAD, CH), f32),
      pltpu.VMEM((ZROWS, CH), f32),
      pltpu.VMEM((B, CH), f32),
      pltpu.VMEM((NBMAX, B), jnp.int32),
      pltpu.VMEM((NBMAX, B), jnp.int32),
  ]
  run = pl.kernel(_sc_body, out_type=out_type, mesh=mesh,
                  scratch_types=scratch_types,
                  compiler_params=pltpu.CompilerParams(
                      use_tc_tiling_on_sc=False))
  return run(*xu_tabs, *xi_tabs, idx_ui[0], idx_ui[1], idx_iu[0], idx_iu[1],
             idx_uu[0], idx_uu[1])


def _tc_body(x_user, x_item, agg_ui, deg_ui, agg_iu, deg_iu, agg_uu, deg_uu,
             wn_ui, wr_ui, b_ui, wn_iu, wr_iu, b_iu, wn_uu, wr_uu, b_uu,
             out_user, out_item):
  dot = functools.partial(jnp.dot, preferred_element_type=jnp.float32)

  def neigh(agg_ref, deg_ref, wn_ref):
    d = jnp.clip(deg_ref[0, :, 0:1] + deg_ref[1, :, 0:1], 1.0)
    m = (agg_ref[0] + agg_ref[1]) / d
    return dot(m, wn_ref[...])

  xu = x_user[...]
  xi = x_item[...]
  out_user[...] = (neigh(agg_iu, deg_iu, wn_iu)
                   + neigh(agg_uu, deg_uu, wn_uu)
                   + dot(xu, wr_iu[...]) + dot(xu, wr_uu[...])
                   + b_iu[...] + b_uu[...])
  out_item[...] = (neigh(agg_ui, deg_ui, wn_ui)
                   + dot(xi, wr_ui[...]) + b_ui[...])


def _tc_combine(x_user, x_item, aggs, weights):
  R = 2000
  grid = (N_USER // R,)
  f32 = jnp.float32

  x_spec = pl.BlockSpec((R, D), lambda i: (i, 0))
  agg_spec = pl.BlockSpec((NC, R, D), lambda i: (0, i, 0))
  deg_spec = pl.BlockSpec((NC, R, CH), lambda i: (0, i, 0))
  w_spec = pl.BlockSpec((D, D), lambda i: (0, 0))
  b_spec = pl.BlockSpec((1, D), lambda i: (0, 0))

  in_specs = [x_spec, x_spec]
  for _ in range(3):
    in_specs += [agg_spec, deg_spec]
  for _ in range(3):
    in_specs += [w_spec, w_spec, b_spec]

  out_user, out_item = pl.pallas_call(
      _tc_body,
      grid=grid,
      in_specs=in_specs,
      out_specs=[x_spec, x_spec],
      out_shape=[jax.ShapeDtypeStruct((N_USER, D), f32),
                 jax.ShapeDtypeStruct((N_ITEM, D), f32)],
  )(x_user, x_item, *aggs, *weights)
  return out_user, out_item


def _prep_idx(edge_index, nb):
  e = edge_index.astype(jnp.int32)
  epad = NW * nb * B
  pad = epad - e.shape[1]
  src = jnp.concatenate([e[0], jnp.zeros((pad,), jnp.int32)])
  dst = jnp.concatenate([e[1], jnp.full((pad,), PAD_DST, jnp.int32)])
  return src.reshape(NW, nb, B), dst.reshape(NW, nb, B)


@jax.jit
def kernel(x_user, x_item, edge_index_user_rates_item,
           edge_index_item_rated_by_user, edge_index_user_follows_user,
           W_neigh_ui, W_root_ui, b_ui, W_neigh_iu, W_root_iu, b_iu,
             W_neigh_uu, W_root_uu, b_uu):
  xu_tabs = [x_user[:, c * CH:(c + 1) * CH] for c in range(NCHUNK)]
  xi_tabs = [x_item[:, c * CH:(c + 1) * CH] for c in range(NCHUNK)]
  idx_ui = _prep_idx(edge_index_user_rates_item, NB_UI)
  idx_iu = _prep_idx(edge_index_item_rated_by_user, NB_UI)
  idx_uu = _prep_idx(edge_index_user_follows_user, NB_UU)

  aggs = _sc_aggregate(xu_tabs, xi_tabs, idx_ui, idx_iu, idx_uu)

  weights = (W_neigh_ui, W_root_ui, b_ui.reshape(1, D),
             W_neigh_iu, W_root_iu, b_iu.reshape(1, D),
             W_neigh_uu, W_root_uu, b_uu.reshape(1, D))
  return _tc_combine(x_user, x_item, aggs, weights)


# speedup vs baseline: 2.0805x; 1.1859x over previous
import functools

import jax
import jax.numpy as jnp
from jax import lax
from jax.experimental import pallas as pl
from jax.experimental.pallas import tpu as pltpu
from jax.experimental.pallas import tpu_sc as plsc

N_USER = 50000
N_ITEM = 50000
D = 128
NC = 2
NS = 16
NW = NC * NS
B = 128
CH = 32
NCHUNK = D // CH
NPAD = 50176
ROWS_PER_TILE = NPAD // NS
ZROWS = 224
NZCOPY = ROWS_PER_TILE // ZROWS
DRAIN = ROWS_PER_TILE // 2
PAD_DST = 50000

E_UI = 320000
E_UU = 160000
NB_UI = 80
NB_UU = 40
NBUF = 4


def _sc_body(xu0, xu1, xu2, xu3, xi0, xi1, xi2, xi3,
             s_ui, d_ui, s_iu, d_iu, s_uu, d_uu,
             agg_ui, deg_ui, agg_iu, deg_iu, agg_uu, deg_uu,
             acc, zbuf, gbuf, sidx, didx,
             g0, g1, g2, g3, s0, s1, s2, s3, bulk):
  gsem = (g0, g1, g2, g3)
  ssem = (s0, s1, s2, s3)
  c = lax.axis_index("c")
  s = lax.axis_index("s")
  w = c * NS + s

  zeros16 = jnp.zeros((16,), jnp.float32)
  onehot16 = jnp.where(lax.iota(jnp.int32, 16) == 0, 1.0, 0.0)

  @pl.loop(0, ZROWS)
  def _(i):
    zbuf[i, pl.ds(0, 16)] = zeros16
    zbuf[i, pl.ds(16, 16)] = zeros16

  def zero_acc():
    cps = [pltpu.async_copy(
        zbuf, acc.at[pl.ds(s * ROWS_PER_TILE + k * ZROWS, ZROWS)], bulk)
        for k in range(NZCOPY)]
    for cp in cps:
      cp.wait()

  zero_acc()
  plsc.subcore_barrier()

  rels = [
      ((xu0, xu1, xu2, xu3), s_ui, d_ui, agg_ui, deg_ui, NB_UI // NBUF),
      ((xi0, xi1, xi2, xi3), s_iu, d_iu, agg_iu, deg_iu, NB_UI // NBUF),
      ((xu0, xu1, xu2, xu3), s_uu, d_uu, agg_uu, deg_uu, NB_UU // NBUF),
  ]

  for rel_i, (tabs, s_hbm, d_hbm, agg_out, deg_out, G) in enumerate(rels):
    for chunk in range(NCHUNK + 1):
      is_deg = chunk == NCHUNK
      last_pass = rel_i == 2 and is_deg
      pe = (G - 1) % 2

      if is_deg:
        @pl.loop(0, B)
        def _(i):
          gbuf[0, i, pl.ds(0, 16)] = onehot16
          gbuf[0, i, pl.ds(16, 16)] = zeros16

        pltpu.sync_copy(d_hbm.at[w, 0], didx.at[0])
        for k in range(NBUF):
          pltpu.async_copy(gbuf.at[0], acc.at[didx.at[0, k]], ssem[k],
                           add=True)

        @pl.loop(0, G - 1)
        def _(g):
          p = lax.rem(g, 2)
          pn = 1 - p
          cpi = pltpu.async_copy(d_hbm.at[w, g + 1], didx.at[pn], bulk)
          cpi.wait()
          for k in range(NBUF):
            pltpu.make_async_copy(gbuf.at[0], acc.at[didx.at[p, k]],
                                  ssem[k]).wait()
            pltpu.async_copy(gbuf.at[0], acc.at[didx.at[pn, k]], ssem[k],
                             add=True)

        for k in range(NBUF):
          pltpu.make_async_copy(gbuf.at[0], acc.at[didx.at[pe, k]],
                                ssem[k]).wait()
      else:
        table = tabs[chunk]
        pltpu.sync_copy(s_hbm.at[w, 0], sidx.at[0])
        pltpu.sync_copy(d_hbm.at[w, 0], didx.at[0])
        for k in range(NBUF):
          pltpu.async_copy(table.at[sidx.at[0, k]], gbuf.at[k], gsem[k])

        @pl.loop(0, G - 1)
        def _(g):
          p = lax.rem(g, 2)
          pn = 1 - p
          ci0 = pltpu.async_copy(s_hbm.at[w, g + 1], sidx.at[pn], bulk)
          ci1 = pltpu.async_copy(d_hbm.at[w, g + 1], didx.at[pn], bulk)
          for k in range(NBUF):
            pltpu.make_async_copy(table.at[sidx.at[p, k]], gbuf.at[k],
                                  gsem[k]).wait()
            pltpu.async_copy(gbuf.at[k], acc.at[didx.at[p, k]], ssem[k],
                             add=True)
          ci0.wait()
          ci1.wait()
          for k in range(NBUF):
            pltpu.make_async_copy(gbuf.at[k], acc.at[didx.at[p, k]],
                                  ssem[k]).wait()
            pltpu.async_copy(table.at[sidx.at[pn, k]], gbuf.at[k], gsem[k])

        for k in range(NBUF):
          pltpu.make_async_copy(table.at[sidx.at[pe, k]], gbuf.at[k],
                                gsem[k]).wait()
          pltpu.async_copy(gbuf.at[k], acc.at[didx.at[pe, k]], ssem[k],
                           add=True)
        for k in range(NBUF):
          pltpu.make_async_copy(gbuf.at[k], acc.at[didx.at[pe, k]],
                                ssem[k]).wait()

      plsc.subcore_barrier()
      dcps = []
      for k in range(2):
        r0 = s * ROWS_PER_TILE + k * DRAIN
        srcref = acc.at[pl.ds(r0, DRAIN)]
        if is_deg:
          dcps.append(pltpu.async_copy(
              srcref, deg_out.at[c, pl.ds(r0, DRAIN)], bulk))
        else:
          dcps.append(pltpu.async_copy(
              srcref, agg_out.at[c, pl.ds(r0, DRAIN), pl.ds(chunk * CH, CH)],
              bulk))
      for cp in dcps:
        cp.wait()
      if not last_pass:
        zero_acc()
      plsc.subcore_barrier()


def _sc_aggregate(xu_tabs, xi_tabs, idx_ui, idx_iu, idx_uu):
  mesh = plsc.VectorSubcoreMesh(core_axis_name="c", subcore_axis_name="s")
  f32 = jnp.float32
  out_type = [
      jax.ShapeDtypeStruct((NC, NPAD, D), f32),
      jax.ShapeDtypeStruct((NC, NPAD, CH), f32),
      jax.ShapeDtypeStruct((NC, NPAD, D), f32),
      jax.ShapeDtypeStruct((NC, NPAD, CH), f32),
      jax.ShapeDtypeStruct((NC, NPAD, D), f32),
      jax.ShapeDtypeStruct((NC, NPAD, CH), f32),
  ]
  scratch_types = [
      pltpu.VMEM_SHARED((NPAD, CH), f32),
      pltpu.VMEM((ZROWS, CH), f32),
      pltpu.VMEM((NBUF, B, CH), f32),
      pltpu.VMEM((2, NBUF, B), jnp.int32),
      pltpu.VMEM((2, NBUF, B), jnp.int32),
  ] + [pltpu.SemaphoreType.DMA] * 9
  run = pl.kernel(_sc_body, out_type=out_type, mesh=mesh,
                  scratch_types=scratch_types,
                  compiler_params=pltpu.CompilerParams(
                      use_tc_tiling_on_sc=False))
  return run(*xu_tabs, *xi_tabs, idx_ui[0], idx_ui[1], idx_iu[0], idx_iu[1],
             idx_uu[0], idx_uu[1])


def _tc_body(x_user, x_item, agg_ui, deg_ui, agg_iu, deg_iu, agg_uu, deg_uu,
             wn_ui, wr_ui, b_ui, wn_iu, wr_iu, b_iu, wn_uu, wr_uu, b_uu,
             out_user, out_item):
  dot = functools.partial(jnp.dot, preferred_element_type=jnp.float32)

  def neigh(agg_ref, deg_ref, wn_ref):
    d = jnp.clip(deg_ref[0, :, 0:1] + deg_ref[1, :, 0:1], 1.0)
    m = (agg_ref[0] + agg_ref[1]) / d
    return dot(m, wn_ref[...])

  xu = x_user[...]
  xi = x_item[...]
  out_user[...] = (neigh(agg_iu, deg_iu, wn_iu)
                   + neigh(agg_uu, deg_uu, wn_uu)
                   + dot(xu, wr_iu[...]) + dot(xu, wr_uu[...])
                   + b_iu[...] + b_uu[...])
  out_item[...] = (neigh(agg_ui, deg_ui, wn_ui)
                   + dot(xi, wr_ui[...]) + b_ui[...])


def _tc_combine(x_user, x_item, aggs, weights):
  R = 2000
  grid = (N_USER // R,)
  f32 = jnp.float32

  x_spec = pl.BlockSpec((R, D), lambda i: (i, 0))
  agg_spec = pl.BlockSpec((NC, R, D), lambda i: (0, i, 0))
  deg_spec = pl.BlockSpec((NC, R, CH), lambda i: (0, i, 0))
  w_spec = pl.BlockSpec((D, D), lambda i: (0, 0))
  b_spec = pl.BlockSpec((1, D), lambda i: (0, 0))

  in_specs = [x_spec, x_spec]
  for _ in range(3):
    in_specs += [agg_spec, deg_spec]
  for _ in range(3):
    in_specs += [w_spec, w_spec, b_spec]

  out_user, out_item = pl.pallas_call(
      _tc_body,
      grid=grid,
      in_specs=in_specs,
      out_specs=[x_spec, x_spec],
      out_shape=[jax.ShapeDtypeStruct((N_USER, D), f32),
                 jax.ShapeDtypeStruct((N_ITEM, D), f32)],
  )(x_user, x_item, *aggs, *weights)
  return out_user, out_item


def _prep_idx(edge_index, nb):
  e = edge_index.astype(jnp.int32)
  epad = NW * nb * B
  pad = epad - e.shape[1]
  src = jnp.concatenate([e[0], jnp.zeros((pad,), jnp.int32)])
  dst = jnp.concatenate([e[1], jnp.full((pad,), PAD_DST, jnp.int32)])
  return (src.reshape(NW, nb // NBUF, NBUF, B),
          dst.reshape(NW, nb // NBUF, NBUF, B))


@jax.jit
def kernel(x_user, x_item, edge_index_user_rates_item,
           edge_index_item_rated_by_user, edge_index_user_follows_user,
           W_neigh_ui, W_root_ui, b_ui, W_neigh_iu, W_root_iu, b_iu,
             W_neigh_uu, W_root_uu, b_uu):
  xu_tabs = [x_user[:, c * CH:(c + 1) * CH] for c in range(NCHUNK)]
  xi_tabs = [x_item[:, c * CH:(c + 1) * CH] for c in range(NCHUNK)]
  idx_ui = _prep_idx(edge_index_user_rates_item, NB_UI)
  idx_iu = _prep_idx(edge_index_item_rated_by_user, NB_UI)
  idx_uu = _prep_idx(edge_index_user_follows_user, NB_UU)

  aggs = _sc_aggregate(xu_tabs, xi_tabs, idx_ui, idx_iu, idx_uu)

  weights = (W_neigh_ui, W_root_ui, b_ui.reshape(1, D),
             W_neigh_iu, W_root_iu, b_iu.reshape(1, D),
             W_neigh_uu, W_root_uu, b_uu.reshape(1, D))
  return _tc_combine(x_user, x_item, aggs, weights)


# speedup vs baseline: 3.2404x; 1.5575x over previous
import functools

import jax
import jax.numpy as jnp
from jax import lax
from jax.experimental import pallas as pl
from jax.experimental.pallas import tpu as pltpu
from jax.experimental.pallas import tpu_sc as plsc

N_USER = 50000
N_ITEM = 50000
D = 128
NC = 2
NS = 16
NW = NC * NS
B = 128
CH = 32
NCHUNK = D // CH
CH2 = 16
NCH2 = D // CH2
NPAD = 50176
ROWS_PER_TILE = NPAD // NS
ZROWS = 224
NZCOPY = ROWS_PER_TILE // ZROWS
DRAIN = ROWS_PER_TILE // 2
PAD_DST = 50000

E_UI = 320000
E_UU = 160000
NB_UI = 80
NB_UU = 40
NBUF = 4


def _sc_body(xpad_u, xpad_i,
             s_ui, d_ui, s_iu, d_iu, s_uu, d_uu,
             agg_ui, deg_ui, agg_iu, deg_iu, agg_uu, deg_uu,
             acc, table, zbuf, gbuf, sidx, didx,
             g0, g1, g2, g3, s0, s1, s2, s3, bulk):
  gsem = (g0, g1, g2, g3)
  ssem = (s0, s1, s2, s3)
  c = lax.axis_index("c")
  s = lax.axis_index("s")
  w = c * NS + s
  r0t = s * ROWS_PER_TILE

  zeros16 = jnp.zeros((16,), jnp.float32)
  onehot16 = jnp.where(lax.iota(jnp.int32, 16) == 0, 1.0, 0.0)

  @pl.loop(0, ZROWS)
  def _(i):
    zbuf[i, pl.ds(0, 16)] = zeros16

  def fill_and_zero(xpad, chunk):
    cps = []
    if xpad is not None:
      cps.append(pltpu.async_copy(
          xpad.at[pl.ds(r0t, ROWS_PER_TILE), pl.ds(chunk * CH2, CH2)],
          table.at[pl.ds(r0t, ROWS_PER_TILE)], bulk))
    cps += [pltpu.async_copy(
        zbuf, acc.at[pl.ds(r0t + k * ZROWS, ZROWS)], bulk)
        for k in range(NZCOPY)]
    for cp in cps:
      cp.wait()

  rels = [
      (xpad_u, s_ui, d_ui, agg_ui, deg_ui, NB_UI // NBUF),
      (xpad_i, s_iu, d_iu, agg_iu, deg_iu, NB_UI // NBUF),
      (xpad_u, s_uu, d_uu, agg_uu, deg_uu, NB_UU // NBUF),
  ]

  for rel_i, (xpad, s_hbm, d_hbm, agg_out, deg_out, G) in enumerate(rels):
    for chunk in range(NCH2 + 1):
      is_deg = chunk == NCH2
      pe = (G - 1) % 2

      fill_and_zero(None if is_deg else xpad, chunk)
      plsc.subcore_barrier()

      if is_deg:
        @pl.loop(0, B)
        def _(i):
          gbuf[0, i, pl.ds(0, 16)] = onehot16

        pltpu.sync_copy(d_hbm.at[w, 0], didx.at[0])
        for k in range(NBUF):
          pltpu.async_copy(gbuf.at[0], acc.at[didx.at[0, k]], ssem[k],
                           add=True)

        @pl.loop(0, G - 1)
        def _(g):
          p = lax.rem(g, 2)
          pn = 1 - p
          cpi = pltpu.async_copy(d_hbm.at[w, g + 1], didx.at[pn], bulk)
          cpi.wait()
          for k in range(NBUF):
            pltpu.make_async_copy(gbuf.at[0], acc.at[didx.at[p, k]],
                                  ssem[k]).wait()
            pltpu.async_copy(gbuf.at[0], acc.at[didx.at[pn, k]], ssem[k],
                             add=True)

        for k in range(NBUF):
          pltpu.make_async_copy(gbuf.at[0], acc.at[didx.at[pe, k]],
                                ssem[k]).wait()
      else:
        pltpu.sync_copy(s_hbm.at[w, 0], sidx.at[0])
        pltpu.sync_copy(d_hbm.at[w, 0], didx.at[0])
        for k in range(NBUF):
          pltpu.async_copy(table.at[sidx.at[0, k]], gbuf.at[k], gsem[k])

        @pl.loop(0, G - 1)
        def _(g):
          p = lax.rem(g, 2)
          pn = 1 - p
          ci0 = pltpu.async_copy(s_hbm.at[w, g + 1], sidx.at[pn], bulk)
          ci1 = pltpu.async_copy(d_hbm.at[w, g + 1], didx.at[pn], bulk)
          for k in range(NBUF):
            pltpu.make_async_copy(table.at[sidx.at[p, k]], gbuf.at[k],
                                  gsem[k]).wait()
            pltpu.async_copy(gbuf.at[k], acc.at[didx.at[p, k]], ssem[k],
                             add=True)
          ci0.wait()
          ci1.wait()
          for k in range(NBUF):
            pltpu.make_async_copy(gbuf.at[k], acc.at[didx.at[p, k]],
                                  ssem[k]).wait()
            pltpu.async_copy(table.at[sidx.at[pn, k]], gbuf.at[k], gsem[k])

        for k in range(NBUF):
          pltpu.make_async_copy(table.at[sidx.at[pe, k]], gbuf.at[k],
                                gsem[k]).wait()
          pltpu.async_copy(gbuf.at[k], acc.at[didx.at[pe, k]], ssem[k],
                           add=True)
        for k in range(NBUF):
          pltpu.make_async_copy(gbuf.at[k], acc.at[didx.at[pe, k]],
                                ssem[k]).wait()

      plsc.subcore_barrier()
      dcps = []
      for k in range(2):
        r0 = r0t + k * DRAIN
        srcref = acc.at[pl.ds(r0, DRAIN)]
        if is_deg:
          dcps.append(pltpu.async_copy(
              srcref, deg_out.at[c, pl.ds(r0, DRAIN)], bulk))
        else:
          dcps.append(pltpu.async_copy(
              srcref, agg_out.at[c, pl.ds(r0, DRAIN),
                                 pl.ds(chunk * CH2, CH2)], bulk))
      for cp in dcps:
        cp.wait()


def _sc_aggregate(xpad_u, xpad_i, idx_ui, idx_iu, idx_uu):
  mesh = plsc.VectorSubcoreMesh(core_axis_name="c", subcore_axis_name="s")
  f32 = jnp.float32
  out_type = [
      jax.ShapeDtypeStruct((NC, NPAD, D), f32),
      jax.ShapeDtypeStruct((NC, NPAD, CH2), f32),
      jax.ShapeDtypeStruct((NC, NPAD, D), f32),
      jax.ShapeDtypeStruct((NC, NPAD, CH2), f32),
      jax.ShapeDtypeStruct((NC, NPAD, D), f32),
      jax.ShapeDtypeStruct((NC, NPAD, CH2), f32),
  ]
  scratch_types = [
      pltpu.VMEM_SHARED((NPAD, CH2), f32),
      pltpu.VMEM_SHARED((NPAD, CH2), f32),
      pltpu.VMEM((ZROWS, CH2), f32),
      pltpu.VMEM((NBUF, B, CH2), f32),
      pltpu.VMEM((2, NBUF, B), jnp.int32),
      pltpu.VMEM((2, NBUF, B), jnp.int32),
  ] + [pltpu.SemaphoreType.DMA] * 9
  run = pl.kernel(_sc_body, out_type=out_type, mesh=mesh,
                  scratch_types=scratch_types,
                  compiler_params=pltpu.CompilerParams(
                      use_tc_tiling_on_sc=False))
  return run(xpad_u, xpad_i, idx_ui[0], idx_ui[1], idx_iu[0], idx_iu[1],
             idx_uu[0], idx_uu[1])


def _tc_body(x_user, x_item, agg_ui, deg_ui, agg_iu, deg_iu, agg_uu, deg_uu,
             wn_ui, wr_ui, b_ui, wn_iu, wr_iu, b_iu, wn_uu, wr_uu, b_uu,
             out_user, out_item):
  dot = functools.partial(jnp.dot, preferred_element_type=jnp.float32)

  def neigh(agg_ref, deg_ref, wn_ref):
    d = jnp.clip(deg_ref[0, :, 0:1] + deg_ref[1, :, 0:1], 1.0)
    m = (agg_ref[0] + agg_ref[1]) / d
    return dot(m, wn_ref[...])

  xu = x_user[...]
  xi = x_item[...]
  out_user[...] = (neigh(agg_iu, deg_iu, wn_iu)
                   + neigh(agg_uu, deg_uu, wn_uu)
                   + dot(xu, wr_iu[...]) + dot(xu, wr_uu[...])
                   + b_iu[...] + b_uu[...])
  out_item[...] = (neigh(agg_ui, deg_ui, wn_ui)
                   + dot(xi, wr_ui[...]) + b_ui[...])


def _tc_combine(x_user, x_item, aggs, weights):
  R = 2000
  grid = (N_USER // R,)
  f32 = jnp.float32

  x_spec = pl.BlockSpec((R, D), lambda i: (i, 0))
  agg_spec = pl.BlockSpec((NC, R, D), lambda i: (0, i, 0))
  deg_spec = pl.BlockSpec((NC, R, CH2), lambda i: (0, i, 0))
  w_spec = pl.BlockSpec((D, D), lambda i: (0, 0))
  b_spec = pl.BlockSpec((1, D), lambda i: (0, 0))

  in_specs = [x_spec, x_spec]
  for _ in range(3):
    in_specs += [agg_spec, deg_spec]
  for _ in range(3):
    in_specs += [w_spec, w_spec, b_spec]

  out_user, out_item = pl.pallas_call(
      _tc_body,
      grid=grid,
      in_specs=in_specs,
      out_specs=[x_spec, x_spec],
      out_shape=[jax.ShapeDtypeStruct((N_USER, D), f32),
                 jax.ShapeDtypeStruct((N_ITEM, D), f32)],
  )(x_user, x_item, *aggs, *weights)
  return out_user, out_item


def _prep_idx(edge_index, nb):
  e = edge_index.astype(jnp.int32)
  epad = NW * nb * B
  pad = epad - e.shape[1]
  src = jnp.concatenate([e[0], jnp.zeros((pad,), jnp.int32)])
  dst = jnp.concatenate([e[1], jnp.full((pad,), PAD_DST, jnp.int32)])
  return (src.reshape(NW, nb // NBUF, NBUF, B),
          dst.reshape(NW, nb // NBUF, NBUF, B))


@jax.jit
def kernel(x_user, x_item, edge_index_user_rates_item,
           edge_index_item_rated_by_user, edge_index_user_follows_user,
           W_neigh_ui, W_root_ui, b_ui, W_neigh_iu, W_root_iu, b_iu,
             W_neigh_uu, W_root_uu, b_uu):
  xpad_u = jnp.pad(x_user, ((0, NPAD - N_USER), (0, 0)))
  xpad_i = jnp.pad(x_item, ((0, NPAD - N_ITEM), (0, 0)))
  idx_ui = _prep_idx(edge_index_user_rates_item, NB_UI)
  idx_iu = _prep_idx(edge_index_item_rated_by_user, NB_UI)
  idx_uu = _prep_idx(edge_index_user_follows_user, NB_UU)

  aggs = _sc_aggregate(xpad_u, xpad_i, idx_ui, idx_iu, idx_uu)

  weights = (W_neigh_ui, W_root_ui, b_ui.reshape(1, D),
             W_neigh_iu, W_root_iu, b_iu.reshape(1, D),
             W_neigh_uu, W_root_uu, b_uu.reshape(1, D))
  return _tc_combine(x_user, x_item, aggs, weights)


# speedup vs baseline: 3.3074x; 1.0207x over previous
import functools

import jax
import jax.numpy as jnp
from jax import lax
from jax.experimental import pallas as pl
from jax.experimental.pallas import tpu as pltpu
from jax.experimental.pallas import tpu_sc as plsc

N_USER = 50000
N_ITEM = 50000
D = 128
NC = 2
NS = 16
NW = NC * NS
B = 128
CH = 32
NCHUNK = D // CH
CH2 = 16
NCH2 = D // CH2
NPAD = 50176
ROWS_PER_TILE = NPAD // NS
ZROWS = 224
NZCOPY = ROWS_PER_TILE // ZROWS
DRAIN = ROWS_PER_TILE // 2
PAD_DST = 50000

E_UI = 320000
E_UU = 160000
NB_UI = 80
NB_UU = 40
NBUF = 4


def _sc_body(xpad_u, xpad_i,
             s_ui, d_ui, s_iu, d_iu, s_uu, d_uu,
             agg_ui, deg_ui, agg_iu, deg_iu, agg_uu, deg_uu,
             acc, table, zbuf, gbuf, sidx, didx,
             g0, g1, g2, g3, s0, s1, s2, s3, bulk):
  gsem = (g0, g1, g2, g3)
  ssem = (s0, s1, s2, s3)
  c = lax.axis_index("c")
  s = lax.axis_index("s")
  w = c * NS + s
  r0t = s * ROWS_PER_TILE

  zeros16 = jnp.zeros((16,), jnp.float32)
  onehot16 = jnp.where(lax.iota(jnp.int32, 16) == 0, 1.0, 0.0)

  @pl.loop(0, ZROWS)
  def _(i):
    zbuf[i, pl.ds(0, 16)] = zeros16

  LAST_ROWS = N_USER - (NS - 1) * ROWS_PER_TILE

  def fill_and_zero(x_hbm, chunk):
    cps = []
    if x_hbm is not None:
      @pl.when(s < NS - 1)
      def _():
        cp = pltpu.async_copy(
            x_hbm.at[pl.ds(r0t, ROWS_PER_TILE), pl.ds(chunk * CH2, CH2)],
            table.at[pl.ds(r0t, ROWS_PER_TILE)], bulk)

      @pl.when(s == NS - 1)
      def _():
        cp = pltpu.async_copy(
            x_hbm.at[pl.ds((NS - 1) * ROWS_PER_TILE, LAST_ROWS),
                     pl.ds(chunk * CH2, CH2)],
            table.at[pl.ds((NS - 1) * ROWS_PER_TILE, LAST_ROWS)], bulk)

    cps += [pltpu.async_copy(
        zbuf, acc.at[pl.ds(r0t + k * ZROWS, ZROWS)], bulk)
        for k in range(NZCOPY)]
    for cp in cps:
      cp.wait()
    if x_hbm is not None:
      @pl.when(s < NS - 1)
      def _():
        pltpu.make_async_copy(
            x_hbm.at[pl.ds(r0t, ROWS_PER_TILE), pl.ds(chunk * CH2, CH2)],
            table.at[pl.ds(r0t, ROWS_PER_TILE)], bulk).wait()

      @pl.when(s == NS - 1)
      def _():
        pltpu.make_async_copy(
            x_hbm.at[pl.ds((NS - 1) * ROWS_PER_TILE, LAST_ROWS),
                     pl.ds(chunk * CH2, CH2)],
            table.at[pl.ds((NS - 1) * ROWS_PER_TILE, LAST_ROWS)],
            bulk).wait()

  rels = [
      (xpad_u, s_ui, d_ui, agg_ui, deg_ui, NB_UI // NBUF),
      (xpad_i, s_iu, d_iu, agg_iu, deg_iu, NB_UI // NBUF),
      (xpad_u, s_uu, d_uu, agg_uu, deg_uu, NB_UU // NBUF),
  ]

  for rel_i, (xpad, s_hbm, d_hbm, agg_out, deg_out, G) in enumerate(rels):
    for chunk in range(NCH2 + 1):
      is_deg = chunk == NCH2
      pe = (G - 1) % 2

      fill_and_zero(None if is_deg else xpad, chunk)
      plsc.subcore_barrier()

      if is_deg:
        @pl.loop(0, B)
        def _(i):
          gbuf[0, i, pl.ds(0, 16)] = onehot16

        pltpu.sync_copy(d_hbm.at[w, 0], didx.at[0])
        for k in range(NBUF):
          pltpu.async_copy(gbuf.at[0], acc.at[didx.at[0, k]], ssem[k],
                           add=True)

        @pl.loop(0, G - 1)
        def _(g):
          p = lax.rem(g, 2)
          pn = 1 - p
          cpi = pltpu.async_copy(d_hbm.at[w, g + 1], didx.at[pn], bulk)
          cpi.wait()
          for k in range(NBUF):
            pltpu.make_async_copy(gbuf.at[0], acc.at[didx.at[p, k]],
                                  ssem[k]).wait()
            pltpu.async_copy(gbuf.at[0], acc.at[didx.at[pn, k]], ssem[k],
                             add=True)

        for k in range(NBUF):
          pltpu.make_async_copy(gbuf.at[0], acc.at[didx.at[pe, k]],
                                ssem[k]).wait()
      else:
        pltpu.sync_copy(s_hbm.at[w, 0], sidx.at[0])
        pltpu.sync_copy(d_hbm.at[w, 0], didx.at[0])
        for k in range(NBUF):
          pltpu.async_copy(table.at[sidx.at[0, k]], gbuf.at[k], gsem[k])

        @pl.loop(0, G - 1)
        def _(g):
          p = lax.rem(g, 2)
          pn = 1 - p
          ci0 = pltpu.async_copy(s_hbm.at[w, g + 1], sidx.at[pn], bulk)
          ci1 = pltpu.async_copy(d_hbm.at[w, g + 1], didx.at[pn], bulk)
          for k in range(NBUF):
            pltpu.make_async_copy(table.at[sidx.at[p, k]], gbuf.at[k],
                                  gsem[k]).wait()
            pltpu.async_copy(gbuf.at[k], acc.at[didx.at[p, k]], ssem[k],
                             add=True)
          ci0.wait()
          ci1.wait()
          for k in range(NBUF):
            pltpu.make_async_copy(gbuf.at[k], acc.at[didx.at[p, k]],
                                  ssem[k]).wait()
            pltpu.async_copy(table.at[sidx.at[pn, k]], gbuf.at[k], gsem[k])

        for k in range(NBUF):
          pltpu.make_async_copy(table.at[sidx.at[pe, k]], gbuf.at[k],
                                gsem[k]).wait()
          pltpu.async_copy(gbuf.at[k], acc.at[didx.at[pe, k]], ssem[k],
                           add=True)
        for k in range(NBUF):
          pltpu.make_async_copy(gbuf.at[k], acc.at[didx.at[pe, k]],
                                ssem[k]).wait()

      plsc.subcore_barrier()
      dcps = []
      for k in range(2):
        r0 = r0t + k * DRAIN
        srcref = acc.at[pl.ds(r0, DRAIN)]
        if is_deg:
          dcps.append(pltpu.async_copy(
              srcref, deg_out.at[c, pl.ds(r0, DRAIN)], bulk))
        else:
          dcps.append(pltpu.async_copy(
              srcref, agg_out.at[c, pl.ds(r0, DRAIN),
                                 pl.ds(chunk * CH2, CH2)], bulk))
      for cp in dcps:
        cp.wait()


def _sc_aggregate(xpad_u, xpad_i, idx_ui, idx_iu, idx_uu):
  mesh = plsc.VectorSubcoreMesh(core_axis_name="c", subcore_axis_name="s")
  f32 = jnp.float32
  out_type = [
      jax.ShapeDtypeStruct((NC, NPAD, D), f32),
      jax.ShapeDtypeStruct((NC, NPAD, CH2), f32),
      jax.ShapeDtypeStruct((NC, NPAD, D), f32),
      jax.ShapeDtypeStruct((NC, NPAD, CH2), f32),
      jax.ShapeDtypeStruct((NC, NPAD, D), f32),
      jax.ShapeDtypeStruct((NC, NPAD, CH2), f32),
  ]
  scratch_types = [
      pltpu.VMEM_SHARED((NPAD, CH2), f32),
      pltpu.VMEM_SHARED((NPAD, CH2), f32),
      pltpu.VMEM((ZROWS, CH2), f32),
      pltpu.VMEM((NBUF, B, CH2), f32),
      pltpu.VMEM((2, NBUF, B), jnp.int32),
      pltpu.VMEM((2, NBUF, B), jnp.int32),
  ] + [pltpu.SemaphoreType.DMA] * 9
  run = pl.kernel(_sc_body, out_type=out_type, mesh=mesh,
                  scratch_types=scratch_types,
                  compiler_params=pltpu.CompilerParams(
                      use_tc_tiling_on_sc=False))
  return run(xpad_u, xpad_i, idx_ui[0], idx_ui[1], idx_iu[0], idx_iu[1],
             idx_uu[0], idx_uu[1])


def _tc_body(x_user, x_item, agg_ui, deg_ui, agg_iu, deg_iu, agg_uu, deg_uu,
             wn_ui, wr_ui, b_ui, wn_iu, wr_iu, b_iu, wn_uu, wr_uu, b_uu,
             out_user, out_item):
  dot = functools.partial(jnp.dot, preferred_element_type=jnp.float32)

  def neigh(agg_ref, deg_ref, wn_ref):
    d = jnp.clip(deg_ref[0, :, 0:1] + deg_ref[1, :, 0:1], 1.0)
    m = (agg_ref[0] + agg_ref[1]) / d
    return dot(m, wn_ref[...])

  xu = x_user[...]
  xi = x_item[...]
  out_user[...] = (neigh(agg_iu, deg_iu, wn_iu)
                   + neigh(agg_uu, deg_uu, wn_uu)
                   + dot(xu, wr_iu[...]) + dot(xu, wr_uu[...])
                   + b_iu[...] + b_uu[...])
  out_item[...] = (neigh(agg_ui, deg_ui, wn_ui)
                   + dot(xi, wr_ui[...]) + b_ui[...])


def _tc_combine(x_user, x_item, aggs, weights):
  R = 2000
  grid = (N_USER // R,)
  f32 = jnp.float32

  x_spec = pl.BlockSpec((R, D), lambda i: (i, 0))
  agg_spec = pl.BlockSpec((NC, R, D), lambda i: (0, i, 0))
  deg_spec = pl.BlockSpec((NC, R, CH2), lambda i: (0, i, 0))
  w_spec = pl.BlockSpec((D, D), lambda i: (0, 0))
  b_spec = pl.BlockSpec((1, D), lambda i: (0, 0))

  in_specs = [x_spec, x_spec]
  for _ in range(3):
    in_specs += [agg_spec, deg_spec]
  for _ in range(3):
    in_specs += [w_spec, w_spec, b_spec]

  out_user, out_item = pl.pallas_call(
      _tc_body,
      grid=grid,
      in_specs=in_specs,
      out_specs=[x_spec, x_spec],
      out_shape=[jax.ShapeDtypeStruct((N_USER, D), f32),
                 jax.ShapeDtypeStruct((N_ITEM, D), f32)],
  )(x_user, x_item, *aggs, *weights)
  return out_user, out_item


def _prep_idx(edge_index, nb):
  e = edge_index.astype(jnp.int32)
  epad = NW * nb * B
  pad = epad - e.shape[1]
  src = jnp.concatenate([e[0], jnp.zeros((pad,), jnp.int32)])
  dst = jnp.concatenate([e[1], jnp.full((pad,), PAD_DST, jnp.int32)])
  return (src.reshape(NW, nb // NBUF, NBUF, B),
          dst.reshape(NW, nb // NBUF, NBUF, B))


@jax.jit
def kernel(x_user, x_item, edge_index_user_rates_item,
           edge_index_item_rated_by_user, edge_index_user_follows_user,
           W_neigh_ui, W_root_ui, b_ui, W_neigh_iu, W_root_iu, b_iu,
             W_neigh_uu, W_root_uu, b_uu):
  idx_ui = _prep_idx(edge_index_user_rates_item, NB_UI)
  idx_iu = _prep_idx(edge_index_item_rated_by_user, NB_UI)
  idx_uu = _prep_idx(edge_index_user_follows_user, NB_UU)

  aggs = _sc_aggregate(x_user, x_item, idx_ui, idx_iu, idx_uu)

  weights = (W_neigh_ui, W_root_ui, b_ui.reshape(1, D),
             W_neigh_iu, W_root_iu, b_iu.reshape(1, D),
             W_neigh_uu, W_root_uu, b_uu.reshape(1, D))
  return _tc_combine(x_user, x_item, aggs, weights)


# speedup vs baseline: 3.4289x; 1.0367x over previous
import functools

import jax
import jax.numpy as jnp
from jax import lax
from jax.experimental import pallas as pl
from jax.experimental.pallas import tpu as pltpu
from jax.experimental.pallas import tpu_sc as plsc

N_USER = 50000
N_ITEM = 50000
D = 128
NC = 2
NS = 16
NW = NC * NS
B = 128
CH = 32
NCHUNK = D // CH
CH2 = 16
NCH2 = D // CH2
NPAD = 50176
ROWS_PER_TILE = NPAD // NS
ZROWS = 224
NZCOPY = ROWS_PER_TILE // ZROWS
DRAIN = ROWS_PER_TILE // 2
PAD_DST = 50000

E_UI = 320000
E_UU = 160000
NB_UI = 80
NB_UU = 40
NBUF = 4


def _make_sc_body(nrel, nbs):
  def body(*refs):
    tables_in = refs[:nrel]
    s_hbms = refs[nrel:2 * nrel]
    d_hbms = refs[2 * nrel:3 * nrel]
    outs = refs[3 * nrel:5 * nrel]
    (acc, table, zbuf, gbuf, sidx, didx,
     g0, g1, g2, g3, s0, s1, s2, s3, bulk) = refs[5 * nrel:]
    gsem = (g0, g1, g2, g3)
    ssem = (s0, s1, s2, s3)
    c = lax.axis_index("c")
    s = lax.axis_index("s")
    w = c * NS + s
    r0t = s * ROWS_PER_TILE

    zeros16 = jnp.zeros((16,), jnp.float32)
    onehot16 = jnp.where(lax.iota(jnp.int32, 16) == 0, 1.0, 0.0)

    @pl.loop(0, ZROWS)
    def _(i):
      zbuf[i, pl.ds(0, 16)] = zeros16

    LAST_ROWS = N_USER - (NS - 1) * ROWS_PER_TILE

    def fill_and_zero(x_hbm, chunk):
      def fill(do_wait):
        op = (pltpu.make_async_copy if do_wait else pltpu.async_copy)

        @pl.when(s < NS - 1)
        def _():
          cp = op(x_hbm.at[pl.ds(r0t, ROWS_PER_TILE),
                           pl.ds(chunk * CH2, CH2)],
                  table.at[pl.ds(r0t, ROWS_PER_TILE)], bulk)
          if do_wait:
            cp.wait()

        @pl.when(s == NS - 1)
        def _():
          cp = op(x_hbm.at[pl.ds((NS - 1) * ROWS_PER_TILE, LAST_ROWS),
                           pl.ds(chunk * CH2, CH2)],
                  table.at[pl.ds((NS - 1) * ROWS_PER_TILE, LAST_ROWS)], bulk)
          if do_wait:
            cp.wait()

      if x_hbm is not None:
        fill(False)
      cps = [pltpu.async_copy(
          zbuf, acc.at[pl.ds(r0t + k * ZROWS, ZROWS)], bulk)
          for k in range(NZCOPY)]
      for cp in cps:
        cp.wait()
      if x_hbm is not None:
        fill(True)

    for rel_i in range(nrel):
      x_hbm = tables_in[rel_i]
      s_hbm = s_hbms[rel_i]
      d_hbm = d_hbms[rel_i]
      agg_out = outs[2 * rel_i]
      deg_out = outs[2 * rel_i + 1]
      G = nbs[rel_i] // NBUF
      for chunk in range(NCH2 + 1):
        is_deg = chunk == NCH2
        pe = (G - 1) % 2

        fill_and_zero(None if is_deg else x_hbm, chunk)
        plsc.subcore_barrier()

        if is_deg:
          @pl.loop(0, B)
          def _(i):
            gbuf[0, i, pl.ds(0, 16)] = onehot16

          pltpu.sync_copy(d_hbm.at[w, 0], didx.at[0])
          for k in range(NBUF):
            pltpu.async_copy(gbuf.at[0], acc.at[didx.at[0, k]], ssem[k],
                             add=True)

          @pl.loop(0, G - 1)
          def _(g):
            p = lax.rem(g, 2)
            pn = 1 - p
            cpi = pltpu.async_copy(d_hbm.at[w, g + 1], didx.at[pn], bulk)
            cpi.wait()
            for k in range(NBUF):
              pltpu.make_async_copy(gbuf.at[0], acc.at[didx.at[p, k]],
                                    ssem[k]).wait()
              pltpu.async_copy(gbuf.at[0], acc.at[didx.at[pn, k]], ssem[k],
                               add=True)

          for k in range(NBUF):
            pltpu.make_async_copy(gbuf.at[0], acc.at[didx.at[pe, k]],
                                  ssem[k]).wait()
        else:
          pltpu.sync_copy(s_hbm.at[w, 0], sidx.at[0])
          pltpu.sync_copy(d_hbm.at[w, 0], didx.at[0])
          for k in range(NBUF):
            pltpu.async_copy(table.at[sidx.at[0, k]], gbuf.at[k], gsem[k])

          @pl.loop(0, G - 1)
          def _(g):
            p = lax.rem(g, 2)
            pn = 1 - p
            ci0 = pltpu.async_copy(s_hbm.at[w, g + 1], sidx.at[pn], bulk)
            ci1 = pltpu.async_copy(d_hbm.at[w, g + 1], didx.at[pn], bulk)
            for k in range(NBUF):
              pltpu.make_async_copy(table.at[sidx.at[p, k]], gbuf.at[k],
                                    gsem[k]).wait()
              pltpu.async_copy(gbuf.at[k], acc.at[didx.at[p, k]], ssem[k],
                               add=True)
            ci0.wait()
            ci1.wait()
            for k in range(NBUF):
              pltpu.make_async_copy(gbuf.at[k], acc.at[didx.at[p, k]],
                                    ssem[k]).wait()
              pltpu.async_copy(table.at[sidx.at[pn, k]], gbuf.at[k], gsem[k])

          for k in range(NBUF):
            pltpu.make_async_copy(table.at[sidx.at[pe, k]], gbuf.at[k],
                                  gsem[k]).wait()
            pltpu.async_copy(gbuf.at[k], acc.at[didx.at[pe, k]], ssem[k],
                             add=True)
          for k in range(NBUF):
            pltpu.make_async_copy(gbuf.at[k], acc.at[didx.at[pe, k]],
                                  ssem[k]).wait()

        plsc.subcore_barrier()
        dcps = []
        for k in range(2):
          r0 = r0t + k * DRAIN
          srcref = acc.at[pl.ds(r0, DRAIN)]
          if is_deg:
            dcps.append(pltpu.async_copy(
                srcref, deg_out.at[c, pl.ds(r0, DRAIN)], bulk))
          else:
            dcps.append(pltpu.async_copy(
                srcref, agg_out.at[c, pl.ds(r0, DRAIN),
                                   pl.ds(chunk * CH2, CH2)], bulk))
        for cp in dcps:
          cp.wait()

  return body


def _sc_aggregate(tables, sidxs, didxs, nbs):
  nrel = len(tables)
  mesh = plsc.VectorSubcoreMesh(core_axis_name="c", subcore_axis_name="s")
  f32 = jnp.float32
  out_type = [jax.ShapeDtypeStruct((NC, NPAD, D), f32),
              jax.ShapeDtypeStruct((NC, NPAD, CH2), f32)] * nrel
  scratch_types = [
      pltpu.VMEM_SHARED((NPAD, CH2), f32),
      pltpu.VMEM_SHARED((NPAD, CH2), f32),
      pltpu.VMEM((ZROWS, CH2), f32),
      pltpu.VMEM((NBUF, B, CH2), f32),
      pltpu.VMEM((2, NBUF, B), jnp.int32),
      pltpu.VMEM((2, NBUF, B), jnp.int32),
  ] + [pltpu.SemaphoreType.DMA] * 9
  run = pl.kernel(_make_sc_body(nrel, tuple(nbs)), out_type=out_type,
                  mesh=mesh, scratch_types=scratch_types,
                  compiler_params=pltpu.CompilerParams(
                      use_tc_tiling_on_sc=False))
  return run(*tables, *sidxs, *didxs)


def _tc_user_body(x_user, agg_iu, deg_iu, agg_uu, deg_uu,
                  wn_iu, wr_iu, b_iu, wn_uu, wr_uu, b_uu, out_user):
  dot = functools.partial(jnp.dot, preferred_element_type=jnp.float32)
  xu = x_user[...]
  out_user[...] = (dot(_mean(agg_iu, deg_iu), wn_iu[...])
                   + dot(_mean(agg_uu, deg_uu), wn_uu[...])
                   + dot(xu, wr_iu[...]) + dot(xu, wr_uu[...])
                   + b_iu[...] + b_uu[...])


def _tc_item_body(x_item, agg_ui, deg_ui, wn_ui, wr_ui, b_ui, out_item):
  dot = functools.partial(jnp.dot, preferred_element_type=jnp.float32)
  out_item[...] = (dot(_mean(agg_ui, deg_ui), wn_ui[...])
                   + dot(x_item[...], wr_ui[...]) + b_ui[...])


def _mean(agg_ref, deg_ref):
  d = jnp.clip(deg_ref[0, :, 0:1] + deg_ref[1, :, 0:1], 1.0)
  return (agg_ref[0] + agg_ref[1]) / d


def _tc_combine(body, x, aggdegs, weights, n_inner):
  R = 2000
  grid = (N_USER // R,)
  f32 = jnp.float32

  x_spec = pl.BlockSpec((R, D), lambda i: (i, 0))
  agg_spec = pl.BlockSpec((NC, R, D), lambda i: (0, i, 0))
  deg_spec = pl.BlockSpec((NC, R, CH2), lambda i: (0, i, 0))
  w_spec = pl.BlockSpec((D, D), lambda i: (0, 0))
  b_spec = pl.BlockSpec((1, D), lambda i: (0, 0))

  in_specs = [x_spec] + [agg_spec, deg_spec] * n_inner
  in_specs += [w_spec, w_spec, b_spec] * n_inner

  return pl.pallas_call(
      body,
      grid=grid,
      in_specs=in_specs,
      out_specs=x_spec,
      out_shape=jax.ShapeDtypeStruct((N_USER, D), f32),
  )(x, *aggdegs, *weights)


def _prep_idx(edge_index, nb):
  e = edge_index.astype(jnp.int32)
  epad = NW * nb * B
  pad = epad - e.shape[1]
  src = jnp.concatenate([e[0], jnp.zeros((pad,), jnp.int32)])
  dst = jnp.concatenate([e[1], jnp.full((pad,), PAD_DST, jnp.int32)])
  return (src.reshape(NW, nb // NBUF, NBUF, B),
          dst.reshape(NW, nb // NBUF, NBUF, B))


@jax.jit
def kernel(x_user, x_item, edge_index_user_rates_item,
           edge_index_item_rated_by_user, edge_index_user_follows_user,
           W_neigh_ui, W_root_ui, b_ui, W_neigh_iu, W_root_iu, b_iu,
           W_neigh_uu, W_root_uu, b_uu):
  idx_ui = _prep_idx(edge_index_user_rates_item, NB_UI)
  idx_iu = _prep_idx(edge_index_item_rated_by_user, NB_UI)
  idx_uu = _prep_idx(edge_index_user_follows_user, NB_UU)

  agg_ui, deg_ui = _sc_aggregate([x_user], [idx_ui[0]], [idx_ui[1]],
                                 [NB_UI])
  agg_iu, deg_iu, agg_uu, deg_uu = _sc_aggregate(
      [x_item, x_user], [idx_iu[0], idx_uu[0]], [idx_iu[1], idx_uu[1]],
      [NB_UI, NB_UU])

  out_item = _tc_combine(
      _tc_item_body, x_item, [agg_ui, deg_ui],
      [W_neigh_ui, W_root_ui, b_ui.reshape(1, D)], 1)
  out_user = _tc_combine(
      _tc_user_body, x_user, [agg_iu, deg_iu, agg_uu, deg_uu],
      [W_neigh_iu, W_root_iu, b_iu.reshape(1, D),
       W_neigh_uu, W_root_uu, b_uu.reshape(1, D)], 2)
  return (out_user, out_item)


# speedup vs baseline: 3.4401x; 1.0033x over previous
import functools

import jax
import jax.numpy as jnp
from jax import lax
from jax.experimental import pallas as pl
from jax.experimental.pallas import tpu as pltpu
from jax.experimental.pallas import tpu_sc as plsc

N_USER = 50000
N_ITEM = 50000
D = 128
NC = 2
NS = 16
NW = NC * NS
B = 128
CH = 32
NCHUNK = D // CH
CH2 = 16
NCH2 = D // CH2
NPAD = 50176
ROWS_PER_TILE = NPAD // NS
ZROWS = 784
NZCOPY = ROWS_PER_TILE // ZROWS
DRAIN = ROWS_PER_TILE // 2
PAD_DST = 50000

E_UI = 320000
E_UU = 160000
NB_UI = 80
NB_UU = 40
NBUF = 4


def _make_sc_body(nrel, nbs):
  def body(*refs):
    tables_in = refs[:nrel]
    s_hbms = refs[nrel:2 * nrel]
    d_hbms = refs[2 * nrel:3 * nrel]
    outs = refs[3 * nrel:5 * nrel]
    (acc, table, zbuf, gbuf, sidx, didx,
     g0, g1, g2, g3, s0, s1, s2, s3, bulk) = refs[5 * nrel:]
    gsem = (g0, g1, g2, g3)
    ssem = (s0, s1, s2, s3)
    c = lax.axis_index("c")
    s = lax.axis_index("s")
    w = c * NS + s
    r0t = s * ROWS_PER_TILE

    zeros16 = jnp.zeros((16,), jnp.float32)
    onehot16 = jnp.where(lax.iota(jnp.int32, 16) == 0, 1.0, 0.0)

    @pl.loop(0, ZROWS)
    def _(i):
      zbuf[i, pl.ds(0, 16)] = zeros16

    LAST_ROWS = N_USER - (NS - 1) * ROWS_PER_TILE

    def fill_and_zero(x_hbm, chunk):
      def fill(do_wait):
        op = (pltpu.make_async_copy if do_wait else pltpu.async_copy)

        @pl.when(s < NS - 1)
        def _():
          cp = op(x_hbm.at[pl.ds(r0t, ROWS_PER_TILE),
                           pl.ds(chunk * CH2, CH2)],
                  table.at[pl.ds(r0t, ROWS_PER_TILE)], bulk)
          if do_wait:
            cp.wait()

        @pl.when(s == NS - 1)
        def _():
          cp = op(x_hbm.at[pl.ds((NS - 1) * ROWS_PER_TILE, LAST_ROWS),
                           pl.ds(chunk * CH2, CH2)],
                  table.at[pl.ds((NS - 1) * ROWS_PER_TILE, LAST_ROWS)], bulk)
          if do_wait:
            cp.wait()

      if x_hbm is not None:
        fill(False)
      cps = [pltpu.async_copy(
          zbuf, acc.at[pl.ds(r0t + k * ZROWS, ZROWS)], bulk)
          for k in range(NZCOPY)]
      for cp in cps:
        cp.wait()
      if x_hbm is not None:
        fill(True)

    for rel_i in range(nrel):
      x_hbm = tables_in[rel_i]
      s_hbm = s_hbms[rel_i]
      d_hbm = d_hbms[rel_i]
      agg_out = outs[2 * rel_i]
      deg_out = outs[2 * rel_i + 1]
      G = nbs[rel_i] // NBUF
      for chunk in range(NCH2 + 1):
        is_deg = chunk == NCH2
        pe = (G - 1) % 2

        fill_and_zero(None if is_deg else x_hbm, chunk)
        plsc.subcore_barrier()

        if is_deg:
          @pl.loop(0, B)
          def _(i):
            gbuf[0, i, pl.ds(0, 16)] = onehot16

          pltpu.sync_copy(d_hbm.at[w, 0], didx.at[0])
          for k in range(NBUF):
            pltpu.async_copy(gbuf.at[0], acc.at[didx.at[0, k]], ssem[k],
                             add=True)

          @pl.loop(0, G - 1)
          def _(g):
            p = lax.rem(g, 2)
            pn = 1 - p
            cpi = pltpu.async_copy(d_hbm.at[w, g + 1], didx.at[pn], bulk)
            cpi.wait()
            for k in range(NBUF):
              pltpu.make_async_copy(gbuf.at[0], acc.at[didx.at[p, k]],
                                    ssem[k]).wait()
              pltpu.async_copy(gbuf.at[0], acc.at[didx.at[pn, k]], ssem[k],
                               add=True)

          for k in range(NBUF):
            pltpu.make_async_copy(gbuf.at[0], acc.at[didx.at[pe, k]],
                                  ssem[k]).wait()
        else:
          cI0 = pltpu.async_copy(s_hbm.at[w, 0], sidx.at[0], bulk)
          cI1 = pltpu.async_copy(d_hbm.at[w, 0], didx.at[0], bulk)
          cI0.wait()
          cI1.wait()
          for k in range(NBUF):
            pltpu.async_copy(table.at[sidx.at[0, k]], gbuf.at[k], gsem[k])

          @pl.loop(0, G - 1)
          def _(g):
            p = lax.rem(g, 2)
            pn = 1 - p
            ci0 = pltpu.async_copy(s_hbm.at[w, g + 1], sidx.at[pn], bulk)
            ci1 = pltpu.async_copy(d_hbm.at[w, g + 1], didx.at[pn], bulk)
            for k in range(NBUF):
              pltpu.make_async_copy(table.at[sidx.at[p, k]], gbuf.at[k],
                                    gsem[k]).wait()
              pltpu.async_copy(gbuf.at[k], acc.at[didx.at[p, k]], ssem[k],
                               add=True)
            ci0.wait()
            ci1.wait()
            for k in range(NBUF):
              pltpu.make_async_copy(gbuf.at[k], acc.at[didx.at[p, k]],
                                    ssem[k]).wait()
              pltpu.async_copy(table.at[sidx.at[pn, k]], gbuf.at[k], gsem[k])

          for k in range(NBUF):
            pltpu.make_async_copy(table.at[sidx.at[pe, k]], gbuf.at[k],
                                  gsem[k]).wait()
            pltpu.async_copy(gbuf.at[k], acc.at[didx.at[pe, k]], ssem[k],
                             add=True)
          for k in range(NBUF):
            pltpu.make_async_copy(gbuf.at[k], acc.at[didx.at[pe, k]],
                                  ssem[k]).wait()

        plsc.subcore_barrier()
        dcps = []
        for k in range(2):
          r0 = r0t + k * DRAIN
          srcref = acc.at[pl.ds(r0, DRAIN)]
          if is_deg:
            dcps.append(pltpu.async_copy(
                srcref, deg_out.at[c, pl.ds(r0, DRAIN)], bulk))
          else:
            dcps.append(pltpu.async_copy(
                srcref, agg_out.at[c, pl.ds(r0, DRAIN),
                                   pl.ds(chunk * CH2, CH2)], bulk))
        for cp in dcps:
          cp.wait()

  return body


def _sc_aggregate(tables, sidxs, didxs, nbs):
  nrel = len(tables)
  mesh = plsc.VectorSubcoreMesh(core_axis_name="c", subcore_axis_name="s")
  f32 = jnp.float32
  out_type = [jax.ShapeDtypeStruct((NC, NPAD, D), f32),
              jax.ShapeDtypeStruct((NC, NPAD, CH2), f32)] * nrel
  scratch_types = [
      pltpu.VMEM_SHARED((NPAD, CH2), f32),
      pltpu.VMEM_SHARED((NPAD, CH2), f32),
      pltpu.VMEM((ZROWS, CH2), f32),
      pltpu.VMEM((NBUF, B, CH2), f32),
      pltpu.VMEM((2, NBUF, B), jnp.int32),
      pltpu.VMEM((2, NBUF, B), jnp.int32),
  ] + [pltpu.SemaphoreType.DMA] * 9
  run = pl.kernel(_make_sc_body(nrel, tuple(nbs)), out_type=out_type,
                  mesh=mesh, scratch_types=scratch_types,
                  compiler_params=pltpu.CompilerParams(
                      use_tc_tiling_on_sc=False))
  return run(*tables, *sidxs, *didxs)


def _tc_user_body(x_user, agg_iu, deg_iu, agg_uu, deg_uu,
                  wn_iu, wr_iu, b_iu, wn_uu, wr_uu, b_uu, out_user):
  dot = functools.partial(jnp.dot, preferred_element_type=jnp.float32)
  xu = x_user[...]
  out_user[...] = (dot(_mean(agg_iu, deg_iu), wn_iu[...])
                   + dot(_mean(agg_uu, deg_uu), wn_uu[...])
                   + dot(xu, wr_iu[...]) + dot(xu, wr_uu[...])
                   + b_iu[...] + b_uu[...])


def _tc_item_body(x_item, agg_ui, deg_ui, wn_ui, wr_ui, b_ui, out_item):
  dot = functools.partial(jnp.dot, preferred_element_type=jnp.float32)
  out_item[...] = (dot(_mean(agg_ui, deg_ui), wn_ui[...])
                   + dot(x_item[...], wr_ui[...]) + b_ui[...])


def _mean(agg_ref, deg_ref):
  d = jnp.clip(deg_ref[0, :, 0:1] + deg_ref[1, :, 0:1], 1.0)
  return (agg_ref[0] + agg_ref[1]) / d


def _tc_combine(body, x, aggdegs, weights, n_inner):
  R = 2000
  grid = (N_USER // R,)
  f32 = jnp.float32

  x_spec = pl.BlockSpec((R, D), lambda i: (i, 0))
  agg_spec = pl.BlockSpec((NC, R, D), lambda i: (0, i, 0))
  deg_spec = pl.BlockSpec((NC, R, CH2), lambda i: (0, i, 0))
  w_spec = pl.BlockSpec((D, D), lambda i: (0, 0))
  b_spec = pl.BlockSpec((1, D), lambda i: (0, 0))

  in_specs = [x_spec] + [agg_spec, deg_spec] * n_inner
  in_specs += [w_spec, w_spec, b_spec] * n_inner

  return pl.pallas_call(
      body,
      grid=grid,
      in_specs=in_specs,
      out_specs=x_spec,
      out_shape=jax.ShapeDtypeStruct((N_USER, D), f32),
  )(x, *aggdegs, *weights)


def _prep_idx(edge_index, nb):
  e = edge_index.astype(jnp.int32)
  epad = NW * nb * B
  pad = epad - e.shape[1]
  src = jnp.concatenate([e[0], jnp.zeros((pad,), jnp.int32)])
  dst = jnp.concatenate([e[1], jnp.full((pad,), PAD_DST, jnp.int32)])
  return (src.reshape(NW, nb // NBUF, NBUF, B),
          dst.reshape(NW, nb // NBUF, NBUF, B))


@jax.jit
def kernel(x_user, x_item, edge_index_user_rates_item,
           edge_index_item_rated_by_user, edge_index_user_follows_user,
           W_neigh_ui, W_root_ui, b_ui, W_neigh_iu, W_root_iu, b_iu,
           W_neigh_uu, W_root_uu, b_uu):
  idx_ui = _prep_idx(edge_index_user_rates_item, NB_UI)
  idx_iu = _prep_idx(edge_index_item_rated_by_user, NB_UI)
  idx_uu = _prep_idx(edge_index_user_follows_user, NB_UU)

  agg_ui, deg_ui = _sc_aggregate([x_user], [idx_ui[0]], [idx_ui[1]],
                                 [NB_UI])
  agg_iu, deg_iu, agg_uu, deg_uu = _sc_aggregate(
      [x_item, x_user], [idx_iu[0], idx_uu[0]], [idx_iu[1], idx_uu[1]],
      [NB_UI, NB_UU])

  out_item = _tc_combine(
      _tc_item_body, x_item, [agg_ui, deg_ui],
      [W_neigh_ui, W_root_ui, b_ui.reshape(1, D)], 1)
  out_user = _tc_combine(
      _tc_user_body, x_user, [agg_iu, deg_iu, agg_uu, deg_uu],
      [W_neigh_iu, W_root_iu, b_iu.reshape(1, D),
       W_neigh_uu, W_root_uu, b_uu.reshape(1, D)], 2)
  return (out_user, out_item)


# speedup vs baseline: 3.4897x; 1.0144x over previous
import functools

import jax
import jax.numpy as jnp
from jax import lax
from jax.experimental import pallas as pl
from jax.experimental.pallas import tpu as pltpu
from jax.experimental.pallas import tpu_sc as plsc

N_USER = 50000
N_ITEM = 50000
D = 128
NC = 2
NS = 16
NW = NC * NS
B = 128
CH = 32
NCHUNK = D // CH
CH2 = 16
NCH2 = D // CH2
NPAD = 50176
ROWS_PER_TILE = NPAD // NS
ZROWS = 784
NZCOPY = ROWS_PER_TILE // ZROWS
DRAIN = ROWS_PER_TILE // 2
PAD_DST = 50000

E_UI = 320000
E_UU = 160000
NB_UI = 80
NB_UU = 40
NBUF = 5


def _make_sc_body(nrel, nbs):
  def body(*refs):
    tables_in = refs[:nrel]
    s_hbms = refs[nrel:2 * nrel]
    d_hbms = refs[2 * nrel:3 * nrel]
    outs = refs[3 * nrel:5 * nrel]
    (acc, table, zbuf, gbuf, sidx, didx,
     g0, g1, g2, g3, g4, s0, s1, s2, s3, s4, bulk) = refs[5 * nrel:]
    gsem = (g0, g1, g2, g3, g4)
    ssem = (s0, s1, s2, s3, s4)
    c = lax.axis_index("c")
    s = lax.axis_index("s")
    w = c * NS + s
    r0t = s * ROWS_PER_TILE

    zeros16 = jnp.zeros((16,), jnp.float32)
    onehot16 = jnp.where(lax.iota(jnp.int32, 16) == 0, 1.0, 0.0)

    @pl.loop(0, ZROWS)
    def _(i):
      zbuf[i, pl.ds(0, 16)] = zeros16

    LAST_ROWS = N_USER - (NS - 1) * ROWS_PER_TILE

    def fill_and_zero(x_hbm, chunk):
      def fill(do_wait):
        op = (pltpu.make_async_copy if do_wait else pltpu.async_copy)

        @pl.when(s < NS - 1)
        def _():
          cp = op(x_hbm.at[pl.ds(r0t, ROWS_PER_TILE),
                           pl.ds(chunk * CH2, CH2)],
                  table.at[pl.ds(r0t, ROWS_PER_TILE)], bulk)
          if do_wait:
            cp.wait()

        @pl.when(s == NS - 1)
        def _():
          cp = op(x_hbm.at[pl.ds((NS - 1) * ROWS_PER_TILE, LAST_ROWS),
                           pl.ds(chunk * CH2, CH2)],
                  table.at[pl.ds((NS - 1) * ROWS_PER_TILE, LAST_ROWS)], bulk)
          if do_wait:
            cp.wait()

      if x_hbm is not None:
        fill(False)
      cps = [pltpu.async_copy(
          zbuf, acc.at[pl.ds(r0t + k * ZROWS, ZROWS)], bulk)
          for k in range(NZCOPY)]
      for cp in cps:
        cp.wait()
      if x_hbm is not None:
        fill(True)

    for rel_i in range(nrel):
      x_hbm = tables_in[rel_i]
      s_hbm = s_hbms[rel_i]
      d_hbm = d_hbms[rel_i]
      agg_out = outs[2 * rel_i]
      deg_out = outs[2 * rel_i + 1]
      G = nbs[rel_i] // NBUF
      for chunk in range(NCH2 + 1):
        is_deg = chunk == NCH2
        pe = (G - 1) % 2

        fill_and_zero(None if is_deg else x_hbm, chunk)
        plsc.subcore_barrier()

        if is_deg:
          @pl.loop(0, B)
          def _(i):
            gbuf[0, i, pl.ds(0, 16)] = onehot16

          pltpu.sync_copy(d_hbm.at[w, 0], didx.at[0])
          for k in range(NBUF):
            pltpu.async_copy(gbuf.at[0], acc.at[didx.at[0, k]], ssem[k],
                             add=True)

          @pl.loop(0, G - 1)
          def _(g):
            p = lax.rem(g, 2)
            pn = 1 - p
            cpi = pltpu.async_copy(d_hbm.at[w, g + 1], didx.at[pn], bulk)
            cpi.wait()
            for k in range(NBUF):
              pltpu.make_async_copy(gbuf.at[0], acc.at[didx.at[p, k]],
                                    ssem[k]).wait()
              pltpu.async_copy(gbuf.at[0], acc.at[didx.at[pn, k]], ssem[k],
                               add=True)

          for k in range(NBUF):
            pltpu.make_async_copy(gbuf.at[0], acc.at[didx.at[pe, k]],
                                  ssem[k]).wait()
        else:
          cI0 = pltpu.async_copy(s_hbm.at[w, 0], sidx.at[0], bulk)
          cI1 = pltpu.async_copy(d_hbm.at[w, 0], didx.at[0], bulk)
          cI0.wait()
          cI1.wait()
          for k in range(NBUF):
            pltpu.async_copy(table.at[sidx.at[0, k]], gbuf.at[k], gsem[k])

          @pl.loop(0, G - 1)
          def _(g):
            p = lax.rem(g, 2)
            pn = 1 - p
            ci0 = pltpu.async_copy(s_hbm.at[w, g + 1], sidx.at[pn], bulk)
            ci1 = pltpu.async_copy(d_hbm.at[w, g + 1], didx.at[pn], bulk)
            for k in range(NBUF):
              pltpu.make_async_copy(table.at[sidx.at[p, k]], gbuf.at[k],
                                    gsem[k]).wait()
              pltpu.async_copy(gbuf.at[k], acc.at[didx.at[p, k]], ssem[k],
                               add=True)
            ci0.wait()
            ci1.wait()
            for k in range(NBUF):
              pltpu.make_async_copy(gbuf.at[k], acc.at[didx.at[p, k]],
                                    ssem[k]).wait()
              pltpu.async_copy(table.at[sidx.at[pn, k]], gbuf.at[k], gsem[k])

          for k in range(NBUF):
            pltpu.make_async_copy(table.at[sidx.at[pe, k]], gbuf.at[k],
                                  gsem[k]).wait()
            pltpu.async_copy(gbuf.at[k], acc.at[didx.at[pe, k]], ssem[k],
                             add=True)
          for k in range(NBUF):
            pltpu.make_async_copy(gbuf.at[k], acc.at[didx.at[pe, k]],
                                  ssem[k]).wait()

        plsc.subcore_barrier()
        dcps = []
        for k in range(2):
          r0 = r0t + k * DRAIN
          srcref = acc.at[pl.ds(r0, DRAIN)]
          if is_deg:
            dcps.append(pltpu.async_copy(
                srcref, deg_out.at[c, pl.ds(r0, DRAIN)], bulk))
          else:
            dcps.append(pltpu.async_copy(
                srcref, agg_out.at[c, pl.ds(r0, DRAIN),
                                   pl.ds(chunk * CH2, CH2)], bulk))
        for cp in dcps:
          cp.wait()

  return body


def _sc_aggregate(tables, sidxs, didxs, nbs):
  nrel = len(tables)
  mesh = plsc.VectorSubcoreMesh(core_axis_name="c", subcore_axis_name="s")
  f32 = jnp.float32
  out_type = [jax.ShapeDtypeStruct((NC, NPAD, D), f32),
              jax.ShapeDtypeStruct((NC, NPAD, CH2), f32)] * nrel
  scratch_types = [
      pltpu.VMEM_SHARED((NPAD, CH2), f32),
      pltpu.VMEM_SHARED((NPAD, CH2), f32),
      pltpu.VMEM((ZROWS, CH2), f32),
      pltpu.VMEM((NBUF, B, CH2), f32),
      pltpu.VMEM((2, NBUF, B), jnp.int32),
      pltpu.VMEM((2, NBUF, B), jnp.int32),
  ] + [pltpu.SemaphoreType.DMA] * 11
  run = pl.kernel(_make_sc_body(nrel, tuple(nbs)), out_type=out_type,
                  mesh=mesh, scratch_types=scratch_types,
                  compiler_params=pltpu.CompilerParams(
                      use_tc_tiling_on_sc=False))
  return run(*tables, *sidxs, *didxs)


def _tc_user_body(x_user, agg_iu, deg_iu, agg_uu, deg_uu,
                  wn_iu, wr_iu, b_iu, wn_uu, wr_uu, b_uu, out_user):
  dot = functools.partial(jnp.dot, preferred_element_type=jnp.float32)
  xu = x_user[...]
  out_user[...] = (dot(_mean(agg_iu, deg_iu), wn_iu[...])
                   + dot(_mean(agg_uu, deg_uu), wn_uu[...])
                   + dot(xu, wr_iu[...]) + dot(xu, wr_uu[...])
                   + b_iu[...] + b_uu[...])


def _tc_item_body(x_item, agg_ui, deg_ui, wn_ui, wr_ui, b_ui, out_item):
  dot = functools.partial(jnp.dot, preferred_element_type=jnp.float32)
  out_item[...] = (dot(_mean(agg_ui, deg_ui), wn_ui[...])
                   + dot(x_item[...], wr_ui[...]) + b_ui[...])


def _mean(agg_ref, deg_ref):
  d = jnp.clip(deg_ref[0, :, 0:1] + deg_ref[1, :, 0:1], 1.0)
  return (agg_ref[0] + agg_ref[1]) / d


def _tc_combine(body, x, aggdegs, weights, n_inner):
  R = 2000
  grid = (N_USER // R,)
  f32 = jnp.float32

  x_spec = pl.BlockSpec((R, D), lambda i: (i, 0))
  agg_spec = pl.BlockSpec((NC, R, D), lambda i: (0, i, 0))
  deg_spec = pl.BlockSpec((NC, R, CH2), lambda i: (0, i, 0))
  w_spec = pl.BlockSpec((D, D), lambda i: (0, 0))
  b_spec = pl.BlockSpec((1, D), lambda i: (0, 0))

  in_specs = [x_spec] + [agg_spec, deg_spec] * n_inner
  in_specs += [w_spec, w_spec, b_spec] * n_inner

  return pl.pallas_call(
      body,
      grid=grid,
      in_specs=in_specs,
      out_specs=x_spec,
      out_shape=jax.ShapeDtypeStruct((N_USER, D), f32),
  )(x, *aggdegs, *weights)


def _prep_idx(edge_index, nb):
  e = edge_index.astype(jnp.int32)
  epad = NW * nb * B
  pad = epad - e.shape[1]
  src = jnp.concatenate([e[0], jnp.zeros((pad,), jnp.int32)])
  dst = jnp.concatenate([e[1], jnp.full((pad,), PAD_DST, jnp.int32)])
  return (src.reshape(NW, nb // NBUF, NBUF, B),
          dst.reshape(NW, nb // NBUF, NBUF, B))


@jax.jit
def kernel(x_user, x_item, edge_index_user_rates_item,
           edge_index_item_rated_by_user, edge_index_user_follows_user,
           W_neigh_ui, W_root_ui, b_ui, W_neigh_iu, W_root_iu, b_iu,
           W_neigh_uu, W_root_uu, b_uu):
  idx_ui = _prep_idx(edge_index_user_rates_item, NB_UI)
  idx_iu = _prep_idx(edge_index_item_rated_by_user, NB_UI)
  idx_uu = _prep_idx(edge_index_user_follows_user, NB_UU)

  agg_ui, deg_ui = _sc_aggregate([x_user], [idx_ui[0]], [idx_ui[1]],
                                 [NB_UI])
  agg_iu, deg_iu, agg_uu, deg_uu = _sc_aggregate(
      [x_item, x_user], [idx_iu[0], idx_uu[0]], [idx_iu[1], idx_uu[1]],
      [NB_UI, NB_UU])

  out_item = _tc_combine(
      _tc_item_body, x_item, [agg_ui, deg_ui],
      [W_neigh_ui, W_root_ui, b_ui.reshape(1, D)], 1)
  out_user = _tc_combine(
      _tc_user_body, x_user, [agg_iu, deg_iu, agg_uu, deg_uu],
      [W_neigh_iu, W_root_iu, b_iu.reshape(1, D),
       W_neigh_uu, W_root_uu, b_uu.reshape(1, D)], 2)
  return (out_user, out_item)
